# baseline (speedup 1.0000x reference)
import functools

import jax
import jax.numpy as jnp
from jax import lax
from jax.experimental import pallas as pl
from jax.experimental.pallas import tpu as pltpu
from jax.experimental.pallas import tpu_sc as plsc

_EPS = 1e-5
_NC = 2
_NS = 16
_NW = _NC * _NS
_STAGE_ROWS = 64


def _ln_table(table_ref, gamma_ref, beta_ref, out_ref):
    t = table_ref[...]
    mean = jnp.mean(t, axis=1, keepdims=True)
    cen = t - mean
    var = jnp.mean(cen * cen, axis=1, keepdims=True)
    out_ref[...] = cen * lax.rsqrt(var + _EPS) * gamma_ref[...] + beta_ref[...]


def kernel(task_id, batch_size, table, gamma, beta):
    V, D = table.shape
    B = task_id.shape[0]

    normed = pl.pallas_call(
        _ln_table,
        out_shape=jax.ShapeDtypeStruct((V, D), jnp.float32),
    )(table, gamma.reshape(1, D), beta.reshape(1, D))

    rows_per_w = B // _NW
    chunks = (16, 32, 80, 128, 128, 128)
    offs = (0, 16, 48, 128, 256, 384)
    n_chunks = len(chunks)
    idx_flat = task_id.astype(jnp.int32)
    full_stage_tiles = V // _STAGE_ROWS
    tail_rows = V - full_stage_tiles * _STAGE_ROWS

    mesh = plsc.VectorSubcoreMesh(core_axis_name="c", subcore_axis_name="s")

    @functools.partial(
        pl.kernel,
        mesh=mesh,
        out_type=jax.ShapeDtypeStruct((B, D), jnp.float32),
        scratch_types=[
            pltpu.VMEM((rows_per_w,), jnp.int32),
            pltpu.VMEM((rows_per_w, D), jnp.float32),
            pltpu.VMEM_SHARED((V, D), jnp.float32),
            pltpu.SemaphoreType.DMA,
            pltpu.SemaphoreType.DMA,
            pltpu.SemaphoreType.DMA,
            pltpu.SemaphoreType.DMA,
        ],
    )
    def _gather(idx_hbm, tab_hbm, out_hbm, idx_v, rows_v, shared_tab,
                g0, g1, w0, w1):
        sid = lax.axis_index("s")
        wid = sid * _NC + lax.axis_index("c")
        base = wid * rows_per_w
        gsem = (g0, g1)
        wsem = (w0, w1)
        pltpu.sync_copy(idx_hbm.at[pl.ds(base, rows_per_w)], idx_v)

        def fire_gather(c, src):
            return pltpu.async_copy(
                src.at[idx_v.at[pl.ds(offs[c], chunks[c])]],
                rows_v.at[pl.ds(offs[c], chunks[c])],
                gsem[c % 2],
            )

        def fire_wb(c):
            return pltpu.async_copy(
                rows_v.at[pl.ds(offs[c], chunks[c])],
                out_hbm.at[pl.ds(base + offs[c], chunks[c])],
                wsem[c % 2],
            )

        @pl.when(sid < full_stage_tiles)
        def _stage():
            r0 = sid * _STAGE_ROWS
            pltpu.sync_copy(
                tab_hbm.at[pl.ds(r0, _STAGE_ROWS)],
                shared_tab.at[pl.ds(r0, _STAGE_ROWS)],
            )

        @pl.when(sid == full_stage_tiles)
        def _stage_tail():
            r0 = full_stage_tiles * _STAGE_ROWS
            pltpu.sync_copy(
                tab_hbm.at[pl.ds(r0, tail_rows)],
                shared_tab.at[pl.ds(r0, tail_rows)],
            )

        plsc.subcore_barrier()

        gathers = [fire_gather(0, shared_tab), fire_gather(1, shared_tab)]
        gathers += [None] * (n_chunks - 2)
        wbs = [None] * n_chunks
        for c in range(n_chunks):
            gathers[c].wait()
            if c + 2 < n_chunks:
                gathers[c + 2] = fire_gather(c + 2, shared_tab)
            if c - 2 >= 0:
                wbs[c - 2].wait()
            wbs[c] = fire_wb(c)
        for c in range(max(n_chunks - 2, 0), n_chunks):
            wbs[c].wait()

    return _gather(idx_flat, normed)


# speedup vs baseline: 1.0108x; 1.0108x over previous
import functools

import jax
import jax.numpy as jnp
from jax import lax
from jax.experimental import pallas as pl
from jax.experimental.pallas import tpu as pltpu
from jax.experimental.pallas import tpu_sc as plsc

_EPS = 1e-5
_NC = 2
_NS = 16
_NW = _NC * _NS
_STAGE_ROWS = 64


def _ln_table(table_ref, gamma_ref, beta_ref, out_ref):
    t = table_ref[...]
    mean = jnp.mean(t, axis=1, keepdims=True)
    cen = t - mean
    var = jnp.mean(cen * cen, axis=1, keepdims=True)
    out_ref[...] = cen * lax.rsqrt(var + _EPS) * gamma_ref[...] + beta_ref[...]


def kernel(task_id, batch_size, table, gamma, beta):
    V, D = table.shape
    B = task_id.shape[0]

    normed = pl.pallas_call(
        _ln_table,
        out_shape=jax.ShapeDtypeStruct((V, D), jnp.float32),
    )(table, gamma.reshape(1, D), beta.reshape(1, D))

    rows_per_w = B // _NW
    chunks = (32, 96, 128, 128, 128)
    offs = (0, 32, 128, 256, 384)
    n_chunks = len(chunks)
    idx_flat = task_id.astype(jnp.int32)
    full_stage_tiles = V // _STAGE_ROWS
    tail_rows = V - full_stage_tiles * _STAGE_ROWS

    mesh = plsc.VectorSubcoreMesh(core_axis_name="c", subcore_axis_name="s")

    @functools.partial(
        pl.kernel,
        mesh=mesh,
        out_type=jax.ShapeDtypeStruct((B, D), jnp.float32),
        scratch_types=[
            pltpu.VMEM((rows_per_w,), jnp.int32),
            pltpu.VMEM((rows_per_w, D), jnp.float32),
            pltpu.VMEM_SHARED((V, D), jnp.float32),
            pltpu.SemaphoreType.DMA,
            pltpu.SemaphoreType.DMA,
            pltpu.SemaphoreType.DMA,
            pltpu.SemaphoreType.DMA,
        ],
    )
    def _gather(idx_hbm, tab_hbm, out_hbm, idx_v, rows_v, shared_tab,
                g0, g1, w0, w1):
        sid = lax.axis_index("s")
        wid = sid * _NC + lax.axis_index("c")
        base = wid * rows_per_w
        gsem = (g0, g1)
        wsem = (w0, w1)
        pltpu.sync_copy(idx_hbm.at[pl.ds(base, rows_per_w)], idx_v)

        def fire_gather(c, src):
            return pltpu.async_copy(
                src.at[idx_v.at[pl.ds(offs[c], chunks[c])]],
                rows_v.at[pl.ds(offs[c], chunks[c])],
                gsem[c % 2],
            )

        def fire_wb(c):
            return pltpu.async_copy(
                rows_v.at[pl.ds(offs[c], chunks[c])],
                out_hbm.at[pl.ds(base + offs[c], chunks[c])],
                wsem[c % 2],
            )

        @pl.when(sid < full_stage_tiles)
        def _stage():
            r0 = sid * _STAGE_ROWS
            pltpu.sync_copy(
                tab_hbm.at[pl.ds(r0, _STAGE_ROWS)],
                shared_tab.at[pl.ds(r0, _STAGE_ROWS)],
            )

        @pl.when(sid == full_stage_tiles)
        def _stage_tail():
            r0 = full_stage_tiles * _STAGE_ROWS
            pltpu.sync_copy(
                tab_hbm.at[pl.ds(r0, tail_rows)],
                shared_tab.at[pl.ds(r0, tail_rows)],
            )

        plsc.subcore_barrier()

        gathers = [fire_gather(0, shared_tab), fire_gather(1, shared_tab)]
        gathers += [None] * (n_chunks - 2)
        wbs = [None] * n_chunks
        for c in range(n_chunks):
            gathers[c].wait()
            if c + 2 < n_chunks:
                gathers[c + 2] = fire_gather(c + 2, shared_tab)
            if c - 2 >= 0:
                wbs[c - 2].wait()
            wbs[c] = fire_wb(c)
        for c in range(max(n_chunks - 2, 0), n_chunks):
            wbs[c].wait()

    return _gather(idx_flat, normed)
